# baseline (speedup 1.0000x reference)
import jax
import jax.numpy as jnp
from jax.experimental import pallas as pl
from jax.experimental.pallas import tpu as pltpu

B, F, D, H, DH = 256, 128, 64, 3, 16
CBS, SBS, NSEL, WIN = 16, 16, 2, 8
INNER = H * DH
NB = F // CBS
BT = 32
SCALE = DH ** -0.5
NEG = -1e30


def _ln(x, g, b):
    m = jnp.mean(x, axis=-1, keepdims=True)
    v = jnp.mean((x - m) * (x - m), axis=-1, keepdims=True)
    return (x - m) * jax.lax.rsqrt(v + 1e-5) * g + b


def _msoftmax(s, mask):
    s = jnp.where(mask, s, NEG)
    m = jnp.max(s, axis=-1, keepdims=True)
    e = jnp.exp(s - m)
    return e / jnp.sum(e, axis=-1, keepdims=True)


def _dot_t(a, b):
    return jax.lax.dot_general(a, b, (((1,), (1,)), ((), ())))


def _fwd(x_ref, We_ref, be_ref, Wq_ref, Wk_ref, Wv_ref, Wck_ref, Wcv_ref,
         Wg_ref, bg_ref, Wo_ref, bo_ref, ln1g_ref, ln1b_ref,
         Wt1_ref, bt1_ref, Wt2_ref, bt2_ref, ln2g_ref, ln2b_ref,
         Wf1_ref, bf1_ref, Wf2_ref, bf2_ref, Wh1_ref, bh1_ref,
         Wh2_ref, bh2_ref, o_ref, emb_s, qa_s, ka_s, va_s, oc_s, os_s, ow_s):
    xb = x_ref[...]
    we = We_ref[...]
    be = be_ref[...]
    emb_s[...] = xb * we + be
    embf = emb_s[...]

    Wq = Wq_ref[...]
    Wk = Wk_ref[...]
    Wv = Wv_ref[...]
    Wck = Wck_ref[...]
    Wcv = Wcv_ref[...]

    bi = jax.lax.broadcasted_iota(jnp.int32, (F, H * F), 0)
    bj = jax.lax.broadcasted_iota(jnp.int32, (F, H * F), 1)
    band3 = (jnp.abs(bi - bj % F) <= WIN).astype(jnp.float32)
    er = jax.lax.broadcasted_iota(jnp.int32, (H * NB, H * F), 0)
    ec = jax.lax.broadcasted_iota(jnp.int32, (H * NB, H * F), 1)
    Ebd = ((er // NB == ec // F) & ((ec % F) // CBS == er % NB)).astype(jnp.float32)
    lm = jax.lax.broadcasted_iota(jnp.int32, (1, INNER), 1) // DH
    M_h = [(lm == h).astype(jnp.float32) for h in range(H)]
    HID = (jax.lax.broadcasted_iota(jnp.int32, (H * F, H), 0) // F
           == jax.lax.broadcasted_iota(jnp.int32, (H * F, H), 1)
           ).astype(jnp.float32)
    Wck3 = jnp.concatenate(
        [jnp.pad(Wck, ((0, 0), (h * DH, INNER - (h + 1) * DH))) for h in range(H)], axis=0)
    Wcv3 = jnp.concatenate(
        [jnp.pad(Wcv, ((0, 0), (h * DH, INNER - (h + 1) * DH))) for h in range(H)], axis=0)
    gr = jax.lax.broadcasted_iota(jnp.int32, (3 * H, INNER), 0)
    gc = jax.lax.broadcasted_iota(jnp.int32, (3 * H, INNER), 1)
    R = [(gr == 3 * (gc // DH) + s).astype(jnp.float32) for s in range(3)]
    X3 = (jax.lax.broadcasted_iota(jnp.int32, (H, INNER), 0)
          == jax.lax.broadcasted_iota(jnp.int32, (H, INNER), 1) // DH
          ).astype(jnp.float32)

    qa_s[...] = (embf @ Wq) * SCALE
    ka_s[...] = embf @ Wk
    va_s[...] = embf @ Wv
    kp = jnp.mean(ka_s[...].reshape(BT * NB, CBS, INNER), axis=1)
    vp = jnp.mean(va_s[...].reshape(BT * NB, CBS, INNER), axis=1)
    km_all = kp @ Wck3
    vm_all = vp @ Wcv3
    for b in range(BT):
        r = slice(b * F, (b + 1) * F)
        rb8 = slice(b * NB, (b + 1) * NB)
        qro, Kc = qa_s[r, :], ka_s[r, :]
        Vc = va_s[r, :]
        Kbd = jnp.concatenate([Kc * m for m in M_h], axis=0)
        Vbd = jnp.concatenate([Vc * m for m in M_h], axis=0)
        km = km_all[rb8, :]
        vm = vm_all[rb8, :]
        km3 = jnp.concatenate([km * m for m in M_h], axis=0)
        vm3 = jnp.concatenate([vm * m for m in M_h], axis=0)
        scT = _dot_t(km3, qro).reshape(H, NB, F)
        m1 = jnp.max(scT, axis=1, keepdims=True)
        e = jnp.exp(scT - m1)
        pT = (e / jnp.sum(e, axis=1, keepdims=True)).reshape(H * NB, F)
        oc = jax.lax.dot_general(pT, vm3, (((0,), (0,)), ((), ())))
        i83 = jax.lax.broadcasted_iota(jnp.int32, (H, NB, F), 1)
        idx1 = jnp.min(jnp.where(scT == m1, i83, NB), axis=1, keepdims=True)
        sc2 = jnp.where(i83 == idx1, -3e38, scT)
        m2 = jnp.max(sc2, axis=1, keepdims=True)
        idx2 = jnp.min(jnp.where(sc2 == m2, i83, NB), axis=1, keepdims=True)
        blkm = ((i83 == idx1) | (i83 == idx2)).astype(jnp.float32).reshape(H * NB, F)
        tok = jax.lax.dot_general(blkm, Ebd, (((0,), (0,)), ((), ())))
        sf = _dot_t(qro, Kbd)
        eh = jnp.exp(sf)
        ES = eh * tok
        EW = eh * band3
        P2 = jnp.concatenate([ES, EW], axis=0)
        O2f = P2 @ jnp.concatenate([Vbd, HID], axis=1)
        O2 = O2f[:, :INNER] * ((1.0 / O2f[:, INNER:]) @ X3)
        os_s[r, :] = O2[:F]
        ow_s[r, :] = O2[F:]
        oc_s[r, :] = oc
    gf = jax.nn.sigmoid(emb_s[...] @ Wg_ref[...] + bg_ref[...])
    gated = ((gf @ R[0]) * oc_s[...] + (gf @ R[1]) * os_s[...]
             + (gf @ R[2]) * ow_s[...])
    x1 = (gated @ Wo_ref[...] + bo_ref[...]).reshape(BT, F, D)

    emb3 = emb_s[...].reshape(BT, F, D)
    t = _ln(emb3, ln1g_ref[...], ln1b_ref[...])
    tn = jnp.swapaxes(t, 1, 2).reshape(BT * D, F)
    tz = jax.nn.gelu(tn @ Wt1_ref[...] + bt1_ref[...]) @ Wt2_ref[...] + bt2_ref[...]
    h1 = emb3 + jnp.swapaxes(tz.reshape(BT, D, F), 1, 2)
    un = _ln(h1, ln2g_ref[...], ln2b_ref[...]).reshape(BT * F, D)
    u = jax.nn.gelu(un @ Wf1_ref[...] + bf1_ref[...]) @ Wf2_ref[...] + bf2_ref[...]
    x2 = h1 + u.reshape(BT, F, D)

    pooled = jnp.mean(x1 + x2, axis=1)
    out = jax.nn.gelu(pooled @ Wh1_ref[...] + bh1_ref[...]) @ Wh2_ref[...] + bh2_ref[...]
    o_ref[...] = out


def kernel(x, We, be, Wq, Wk, Wv, Wck, Wcv, Wg, bg, Wo, bo, ln1_g, ln1_b,
           Wt1, bt1, Wt2, bt2, ln2_g, ln2_b, Wf1, bf1, Wf2, bf2,
           Wh1, bh1, Wh2, bh2):
    args = [
        x.reshape(B * F, 1), We, be.reshape(1, D), Wq, Wk, Wv, Wck, Wcv, Wg,
        bg.reshape(1, 3 * H),
        Wo, bo.reshape(1, D), ln1_g.reshape(1, D), ln1_b.reshape(1, D),
        Wt1, bt1.reshape(1, 256), Wt2, bt2.reshape(1, F),
        ln2_g.reshape(1, D), ln2_b.reshape(1, D),
        Wf1, bf1.reshape(1, 256), Wf2, bf2.reshape(1, D),
        Wh1, bh1.reshape(1, 32), Wh2, bh2.reshape(1, 2),
    ]
    in_specs = [pl.BlockSpec((BT * F, 1), lambda i: (i, 0))]
    for a in args[1:]:
        in_specs.append(pl.BlockSpec(a.shape, lambda i: (0, 0)))
    return pl.pallas_call(
        _fwd,
        grid=(B // BT,),
        in_specs=in_specs,
        out_specs=pl.BlockSpec((BT, 2), lambda i: (i, 0)),
        out_shape=jax.ShapeDtypeStruct((B, 2), jnp.float32),
        scratch_shapes=[
            pltpu.VMEM((BT * F, D), jnp.float32),
            pltpu.VMEM((BT * F, INNER), jnp.float32),
            pltpu.VMEM((BT * F, INNER), jnp.float32),
            pltpu.VMEM((BT * F, INNER), jnp.float32),
            pltpu.VMEM((BT * F, INNER), jnp.float32),
            pltpu.VMEM((BT * F, INNER), jnp.float32),
            pltpu.VMEM((BT * F, INNER), jnp.float32),
        ],
    )(*args)


# speedup vs baseline: 1.1651x; 1.1651x over previous
import jax
import jax.numpy as jnp
from jax.experimental import pallas as pl
from jax.experimental.pallas import tpu as pltpu

B, F, D, H, DH = 256, 128, 64, 3, 16
CBS, SBS, NSEL, WIN = 16, 16, 2, 8
INNER = H * DH
NB = F // CBS
BT = 32
SCALE = DH ** -0.5
NEG = -1e30


def _ln(x, g, b):
    m = jnp.mean(x, axis=-1, keepdims=True)
    v = jnp.mean((x - m) * (x - m), axis=-1, keepdims=True)
    return (x - m) * jax.lax.rsqrt(v + 1e-5) * g + b


def _msoftmax(s, mask):
    s = jnp.where(mask, s, NEG)
    m = jnp.max(s, axis=-1, keepdims=True)
    e = jnp.exp(s - m)
    return e / jnp.sum(e, axis=-1, keepdims=True)


def _dot_t(a, b):
    return jax.lax.dot_general(a, b, (((1,), (1,)), ((), ())))


def _fwd(x_ref, We_ref, be_ref, Wq_ref, Wk_ref, Wv_ref, Wck_ref, Wcv_ref,
         Wg_ref, bg_ref, Wo_ref, bo_ref, ln1g_ref, ln1b_ref,
         Wt1_ref, bt1_ref, Wt2_ref, bt2_ref, ln2g_ref, ln2b_ref,
         Wf1_ref, bf1_ref, Wf2_ref, bf2_ref, Wh1_ref, bh1_ref,
         Wh2_ref, bh2_ref, o_ref, emb_s, qa_s, ka_s, va_s, oc_s, os_s, ow_s):
    xb = x_ref[...]
    we = We_ref[...]
    be = be_ref[...]
    emb_s[...] = xb * we + be
    embf = emb_s[...]

    Wq = Wq_ref[...]
    Wk = Wk_ref[...]
    Wv = Wv_ref[...]
    Wck = Wck_ref[...]
    Wcv = Wcv_ref[...]

    bi = jax.lax.broadcasted_iota(jnp.int32, (F, H * F), 0)
    bj = jax.lax.broadcasted_iota(jnp.int32, (F, H * F), 1)
    band3 = (jnp.abs(bi - bj % F) <= WIN).astype(jnp.float32)
    er = jax.lax.broadcasted_iota(jnp.int32, (H * NB, H * F), 0)
    ec = jax.lax.broadcasted_iota(jnp.int32, (H * NB, H * F), 1)
    Ebd = ((er // NB == ec // F) & ((ec % F) // CBS == er % NB)).astype(jnp.float32)
    lm = jax.lax.broadcasted_iota(jnp.int32, (1, INNER), 1) // DH
    M_h = [(lm == h).astype(jnp.float32) for h in range(H)]
    HID = (jax.lax.broadcasted_iota(jnp.int32, (H * F, H), 0) // F
           == jax.lax.broadcasted_iota(jnp.int32, (H * F, H), 1)
           ).astype(jnp.float32)
    Wck3 = jnp.concatenate(
        [jnp.pad(Wck, ((0, 0), (h * DH, INNER - (h + 1) * DH))) for h in range(H)], axis=0)
    Wcv3 = jnp.concatenate(
        [jnp.pad(Wcv, ((0, 0), (h * DH, INNER - (h + 1) * DH))) for h in range(H)], axis=0)
    gr = jax.lax.broadcasted_iota(jnp.int32, (3 * H, INNER), 0)
    gc = jax.lax.broadcasted_iota(jnp.int32, (3 * H, INNER), 1)
    R = [(gr == 3 * (gc // DH) + s).astype(jnp.float32) for s in range(3)]
    X3 = (jax.lax.broadcasted_iota(jnp.int32, (H, INNER), 0)
          == jax.lax.broadcasted_iota(jnp.int32, (H, INNER), 1) // DH
          ).astype(jnp.float32)

    qa_s[...] = (embf @ Wq) * SCALE
    ka_s[...] = embf @ Wk
    va_s[...] = embf @ Wv
    kp = jnp.mean(ka_s[...].reshape(BT * NB, CBS, INNER), axis=1)
    vp = jnp.mean(va_s[...].reshape(BT * NB, CBS, INNER), axis=1)
    km_all = kp @ Wck3
    vm_all = vp @ Wcv3
    for b in range(BT):
        r = slice(b * F, (b + 1) * F)
        rb8 = slice(b * NB, (b + 1) * NB)
        qro, Kc = qa_s[r, :], ka_s[r, :]
        Vc = va_s[r, :]
        Kbd = jnp.concatenate([Kc * m for m in M_h], axis=0)
        Vbd = jnp.concatenate([Vc * m for m in M_h], axis=0)
        km = km_all[rb8, :]
        vm = vm_all[rb8, :]
        km3 = jnp.concatenate([km * m for m in M_h], axis=0)
        vm3 = jnp.concatenate([vm * m for m in M_h], axis=0)
        scT = _dot_t(km3, qro).reshape(H, NB, F)
        m1 = jnp.max(scT, axis=1, keepdims=True)
        e = jnp.exp(scT - m1)
        pT = (e / jnp.sum(e, axis=1, keepdims=True)).reshape(H * NB, F)
        oc = jax.lax.dot_general(pT, vm3, (((0,), (0,)), ((), ())))
        i83 = jax.lax.broadcasted_iota(jnp.int32, (H, NB, F), 1)
        idx1 = jnp.min(jnp.where(scT == m1, i83, NB), axis=1, keepdims=True)
        sc2 = jnp.where(i83 == idx1, -3e38, scT)
        m2 = jnp.max(sc2, axis=1, keepdims=True)
        idx2 = jnp.min(jnp.where(sc2 == m2, i83, NB), axis=1, keepdims=True)
        blkm = ((i83 == idx1) | (i83 == idx2)).astype(jnp.float32).reshape(H * NB, F)
        tok = jax.lax.dot_general(blkm, Ebd, (((0,), (0,)), ((), ())))
        sf = _dot_t(qro, Kbd)
        eh = jnp.exp(sf)
        ES = eh * tok
        EW = eh * band3
        P2 = jnp.concatenate([ES, EW], axis=0)
        O2 = P2 @ Vbd
        S2 = P2 @ HID
        O2 = O2 * ((1.0 / S2) @ X3)
        os_s[r, :] = O2[:F]
        ow_s[r, :] = O2[F:]
        oc_s[r, :] = oc
    gf = jax.nn.sigmoid(emb_s[...] @ Wg_ref[...] + bg_ref[...])
    gated = ((gf @ R[0]) * oc_s[...] + (gf @ R[1]) * os_s[...]
             + (gf @ R[2]) * ow_s[...])
    x1 = (gated @ Wo_ref[...] + bo_ref[...]).reshape(BT, F, D)

    def _mm(a, w):
        return jnp.dot(a.astype(jnp.bfloat16), w.astype(jnp.bfloat16),
                       preferred_element_type=jnp.float32)
    emb3 = emb_s[...].reshape(BT, F, D)
    t = _ln(emb3, ln1g_ref[...], ln1b_ref[...])
    tn = jnp.swapaxes(t, 1, 2).reshape(BT * D, F)
    tz = _mm(jax.nn.gelu(_mm(tn, Wt1_ref[...]) + bt1_ref[...]), Wt2_ref[...]) + bt2_ref[...]
    h1 = emb3 + jnp.swapaxes(tz.reshape(BT, D, F), 1, 2)
    un = _ln(h1, ln2g_ref[...], ln2b_ref[...]).reshape(BT * F, D)
    u = _mm(jax.nn.gelu(_mm(un, Wf1_ref[...]) + bf1_ref[...]), Wf2_ref[...]) + bf2_ref[...]
    x2 = h1 + u.reshape(BT, F, D)

    pooled = jnp.mean(x1 + x2, axis=1)
    out = jax.nn.gelu(pooled @ Wh1_ref[...] + bh1_ref[...]) @ Wh2_ref[...] + bh2_ref[...]
    o_ref[...] = out


def kernel(x, We, be, Wq, Wk, Wv, Wck, Wcv, Wg, bg, Wo, bo, ln1_g, ln1_b,
           Wt1, bt1, Wt2, bt2, ln2_g, ln2_b, Wf1, bf1, Wf2, bf2,
           Wh1, bh1, Wh2, bh2):
    args = [
        x.reshape(B * F, 1), We, be.reshape(1, D), Wq, Wk, Wv, Wck, Wcv, Wg,
        bg.reshape(1, 3 * H),
        Wo, bo.reshape(1, D), ln1_g.reshape(1, D), ln1_b.reshape(1, D),
        Wt1, bt1.reshape(1, 256), Wt2, bt2.reshape(1, F),
        ln2_g.reshape(1, D), ln2_b.reshape(1, D),
        Wf1, bf1.reshape(1, 256), Wf2, bf2.reshape(1, D),
        Wh1, bh1.reshape(1, 32), Wh2, bh2.reshape(1, 2),
    ]
    in_specs = [pl.BlockSpec((BT * F, 1), lambda i: (i, 0))]
    for a in args[1:]:
        in_specs.append(pl.BlockSpec(a.shape, lambda i: (0, 0)))
    return pl.pallas_call(
        _fwd,
        grid=(B // BT,),
        in_specs=in_specs,
        out_specs=pl.BlockSpec((BT, 2), lambda i: (i, 0)),
        out_shape=jax.ShapeDtypeStruct((B, 2), jnp.float32),
        scratch_shapes=[
            pltpu.VMEM((BT * F, D), jnp.float32),
            pltpu.VMEM((BT * F, INNER), jnp.float32),
            pltpu.VMEM((BT * F, INNER), jnp.float32),
            pltpu.VMEM((BT * F, INNER), jnp.float32),
            pltpu.VMEM((BT * F, INNER), jnp.float32),
            pltpu.VMEM((BT * F, INNER), jnp.float32),
            pltpu.VMEM((BT * F, INNER), jnp.float32),
        ],
    )(*args)


# speedup vs baseline: 1.1925x; 1.0235x over previous
import jax
import jax.numpy as jnp
from jax.experimental import pallas as pl
from jax.experimental.pallas import tpu as pltpu

B, F, D, H, DH = 256, 128, 64, 3, 16
CBS, SBS, NSEL, WIN = 16, 16, 2, 8
INNER = H * DH
NB = F // CBS
BT = 32
SCALE = DH ** -0.5
NEG = -1e30


def _ln(x, g, b):
    m = jnp.mean(x, axis=-1, keepdims=True)
    v = jnp.mean((x - m) * (x - m), axis=-1, keepdims=True)
    return (x - m) * jax.lax.rsqrt(v + 1e-5) * g + b


def _msoftmax(s, mask):
    s = jnp.where(mask, s, NEG)
    m = jnp.max(s, axis=-1, keepdims=True)
    e = jnp.exp(s - m)
    return e / jnp.sum(e, axis=-1, keepdims=True)


def _dot_t(a, b):
    return jax.lax.dot_general(a, b, (((1,), (1,)), ((), ())))


def _fwd(x_ref, We_ref, be_ref, Wq_ref, Wk_ref, Wv_ref, Wck_ref, Wcv_ref,
         Wg_ref, bg_ref, Wo_ref, bo_ref, ln1g_ref, ln1b_ref,
         Wt1_ref, bt1_ref, Wt2_ref, bt2_ref, ln2g_ref, ln2b_ref,
         Wf1_ref, bf1_ref, Wf2_ref, bf2_ref, Wh1_ref, bh1_ref,
         Wh2_ref, bh2_ref, o_ref, emb_s, qa_s, ka_s, va_s, oc_s, os_s, ow_s):
    xb = x_ref[...]
    we = We_ref[...]
    be = be_ref[...]
    emb_s[...] = xb * we + be
    embf = emb_s[...]

    Wq = Wq_ref[...]
    Wk = Wk_ref[...]
    Wv = Wv_ref[...]
    Wck = Wck_ref[...]
    Wcv = Wcv_ref[...]

    bi = jax.lax.broadcasted_iota(jnp.int32, (F, H * F), 0)
    bj = jax.lax.broadcasted_iota(jnp.int32, (F, H * F), 1)
    band3 = (jnp.abs(bi - bj % F) <= WIN).astype(jnp.float32)
    er = jax.lax.broadcasted_iota(jnp.int32, (H * NB, H * F), 0)
    ec = jax.lax.broadcasted_iota(jnp.int32, (H * NB, H * F), 1)
    Ebd = ((er // NB == ec // F) & ((ec % F) // CBS == er % NB)).astype(jnp.float32)
    lm = jax.lax.broadcasted_iota(jnp.int32, (1, INNER), 1) // DH
    M_h = [(lm == h).astype(jnp.float32) for h in range(H)]
    HID = (jax.lax.broadcasted_iota(jnp.int32, (H * F, H), 0) // F
           == jax.lax.broadcasted_iota(jnp.int32, (H * F, H), 1)
           ).astype(jnp.float32)
    Wck3 = jnp.concatenate(
        [jnp.pad(Wck, ((0, 0), (h * DH, INNER - (h + 1) * DH))) for h in range(H)], axis=0)
    Wcv3 = jnp.concatenate(
        [jnp.pad(Wcv, ((0, 0), (h * DH, INNER - (h + 1) * DH))) for h in range(H)], axis=0)
    gr = jax.lax.broadcasted_iota(jnp.int32, (3 * H, INNER), 0)
    gc = jax.lax.broadcasted_iota(jnp.int32, (3 * H, INNER), 1)
    R = [(gr == 3 * (gc // DH) + s).astype(jnp.float32) for s in range(3)]
    X3 = (jax.lax.broadcasted_iota(jnp.int32, (H, INNER), 0)
          == jax.lax.broadcasted_iota(jnp.int32, (H, INNER), 1) // DH
          ).astype(jnp.float32)

    qa_s[...] = (embf @ Wq) * SCALE
    ka_s[...] = embf @ Wk
    va_s[...] = embf @ Wv
    pxb = jnp.mean(xb.reshape(BT * NB, CBS, 1), axis=1)
    pe = pxb * we + be
    km_all = pe @ (Wk @ Wck3)
    vm_all = pe @ (Wv @ Wcv3)
    for b in range(BT):
        r = slice(b * F, (b + 1) * F)
        rb8 = slice(b * NB, (b + 1) * NB)
        qro, Kc = qa_s[r, :], ka_s[r, :]
        Vc = va_s[r, :]
        Kbd = jnp.concatenate([Kc * m for m in M_h], axis=0)
        Vbd = jnp.concatenate([Vc * m for m in M_h], axis=0)
        km = km_all[rb8, :]
        vm = vm_all[rb8, :]
        km3 = jnp.concatenate([km * m for m in M_h], axis=0)
        vm3 = jnp.concatenate([vm * m for m in M_h], axis=0)
        scT = _dot_t(km3, qro).reshape(H, NB, F)
        m1 = jnp.max(scT, axis=1, keepdims=True)
        e = jnp.exp(scT - m1)
        pT = (e / jnp.sum(e, axis=1, keepdims=True)).reshape(H * NB, F)
        oc = jax.lax.dot_general(pT, vm3, (((0,), (0,)), ((), ())))
        i83 = jax.lax.broadcasted_iota(jnp.int32, (H, NB, F), 1)
        idx1 = jnp.min(jnp.where(scT == m1, i83, NB), axis=1, keepdims=True)
        sc2 = jnp.where(i83 == idx1, -3e38, scT)
        m2 = jnp.max(sc2, axis=1, keepdims=True)
        idx2 = jnp.min(jnp.where(sc2 == m2, i83, NB), axis=1, keepdims=True)
        blkm = ((i83 == idx1) | (i83 == idx2)).astype(jnp.float32).reshape(H * NB, F)
        tok = jax.lax.dot_general(blkm, Ebd, (((0,), (0,)), ((), ())))
        sf = _dot_t(qro, Kbd)
        eh = jnp.exp(sf)
        ES = eh * tok
        EW = eh * band3
        P2 = jnp.concatenate([ES, EW], axis=0)
        O2 = P2 @ Vbd
        S2 = P2 @ HID
        O2 = O2 * ((1.0 / S2) @ X3)
        os_s[r, :] = O2[:F]
        ow_s[r, :] = O2[F:]
        oc_s[r, :] = oc
    gf = jax.nn.sigmoid(emb_s[...] @ Wg_ref[...] + bg_ref[...])
    gated = ((gf @ R[0]) * oc_s[...] + (gf @ R[1]) * os_s[...]
             + (gf @ R[2]) * ow_s[...])
    x1 = (gated @ Wo_ref[...] + bo_ref[...]).reshape(BT, F, D)

    emb3 = emb_s[...].reshape(BT, F, D)
    t = _ln(emb3, ln1g_ref[...], ln1b_ref[...])
    tn = jnp.swapaxes(t, 1, 2).reshape(BT * D, F)
    tz = jax.nn.gelu(tn @ Wt1_ref[...] + bt1_ref[...]) @ Wt2_ref[...] + bt2_ref[...]
    h1 = emb3 + jnp.swapaxes(tz.reshape(BT, D, F), 1, 2)
    un = _ln(h1, ln2g_ref[...], ln2b_ref[...]).reshape(BT * F, D)
    u = jax.nn.gelu(un @ Wf1_ref[...] + bf1_ref[...]) @ Wf2_ref[...] + bf2_ref[...]
    x2 = h1 + u.reshape(BT, F, D)

    pooled = jnp.mean(x1 + x2, axis=1)
    out = jax.nn.gelu(pooled @ Wh1_ref[...] + bh1_ref[...]) @ Wh2_ref[...] + bh2_ref[...]
    o_ref[...] = out


def kernel(x, We, be, Wq, Wk, Wv, Wck, Wcv, Wg, bg, Wo, bo, ln1_g, ln1_b,
           Wt1, bt1, Wt2, bt2, ln2_g, ln2_b, Wf1, bf1, Wf2, bf2,
           Wh1, bh1, Wh2, bh2):
    args = [
        x.reshape(B * F, 1), We, be.reshape(1, D), Wq, Wk, Wv, Wck, Wcv, Wg,
        bg.reshape(1, 3 * H),
        Wo, bo.reshape(1, D), ln1_g.reshape(1, D), ln1_b.reshape(1, D),
        Wt1, bt1.reshape(1, 256), Wt2, bt2.reshape(1, F),
        ln2_g.reshape(1, D), ln2_b.reshape(1, D),
        Wf1, bf1.reshape(1, 256), Wf2, bf2.reshape(1, D),
        Wh1, bh1.reshape(1, 32), Wh2, bh2.reshape(1, 2),
    ]
    in_specs = [pl.BlockSpec((BT * F, 1), lambda i: (i, 0))]
    for a in args[1:]:
        in_specs.append(pl.BlockSpec(a.shape, lambda i: (0, 0)))
    return pl.pallas_call(
        _fwd,
        grid=(B // BT,),
        in_specs=in_specs,
        out_specs=pl.BlockSpec((BT, 2), lambda i: (i, 0)),
        out_shape=jax.ShapeDtypeStruct((B, 2), jnp.float32),
        scratch_shapes=[
            pltpu.VMEM((BT * F, D), jnp.float32),
            pltpu.VMEM((BT * F, INNER), jnp.float32),
            pltpu.VMEM((BT * F, INNER), jnp.float32),
            pltpu.VMEM((BT * F, INNER), jnp.float32),
            pltpu.VMEM((BT * F, INNER), jnp.float32),
            pltpu.VMEM((BT * F, INNER), jnp.float32),
            pltpu.VMEM((BT * F, INNER), jnp.float32),
        ],
    )(*args)
